# baseline (speedup 1.0000x reference)
import functools

import jax
import jax.numpy as jnp
from jax import lax
from jax.experimental import pallas as pl
from jax.experimental.pallas import tpu as pltpu
from jax.experimental.pallas import tpu_sc as plsc

_VOCAB = 1000000
_EMBED = 64
_BATCH = 16384
_SEQ = 50

_NC = 2
_NS = 16
_NW = _NC * _NS
_CHUNK = 128
_NBLK = _BATCH // _CHUNK
_UNITS = _SEQ * _NBLK
_UPW = _UNITS // _NW
_NBUF = 4
_NGRP = _UPW // _NBUF

_mesh = plsc.VectorSubcoreMesh(core_axis_name="c", subcore_axis_name="s")


@functools.partial(
    pl.kernel,
    mesh=_mesh,
    out_type=jax.ShapeDtypeStruct((_SEQ, 8, _NBLK, 8, _CHUNK), jnp.float32),
    scratch_types=(
        [pltpu.VMEM((_UPW, _CHUNK), jnp.int32)]
        + [pltpu.VMEM((_CHUNK, _EMBED), jnp.float32)] * _NBUF
        + [pltpu.VMEM((8, 8, _CHUNK), jnp.float32)] * _NBUF
        + [pltpu.SemaphoreType.DMA] * _NBUF
        + [pltpu.SemaphoreType.DMA] * _NBUF
    ),
    compiler_params=pltpu.CompilerParams(use_tc_tiling_on_sc=False,
                                         needs_layout_passes=False),
)
def _sc_gather(idx_hbm, table_hbm, out_hbm, idx_v,
               g0, g1, g2, g3, t0, t1, t2, t3,
               gs0, gs1, gs2, gs3, ss0, ss1, ss2, ss3):
    gbuf = (g0, g1, g2, g3)
    tbuf = (t0, t1, t2, t3)
    gsem = (gs0, gs1, gs2, gs3)
    ssem = (ss0, ss1, ss2, ss3)
    wid = lax.axis_index("s") * _NC + lax.axis_index("c")
    ubase = wid * _UPW

    pltpu.sync_copy(idx_hbm.at[wid], idx_v)

    rowb = [lax.iota(jnp.int32, 16) + (grp * 16) for grp in range(8)]
    zero16 = jnp.zeros((16,), jnp.int32)

    def _start_gather(t, b):
        pltpu.make_async_copy(table_hbm.at[idx_v.at[t]], gbuf[b],
                              gsem[b]).start()

    def _wait_gather(t, b):
        pltpu.make_async_copy(table_hbm.at[idx_v.at[t]], gbuf[b],
                              gsem[b]).wait()

    def _transpose(b):
        def _egrp(i, carry):
            colbase = zero16 + i * 8
            for k in range(8):
                col = colbase + k
                for grp in range(8):
                    v = plsc.load_gather(gbuf[b], [rowb[grp], col])
                    tbuf[b][i, k, pl.ds(grp * 16, 16)] = v
            return carry
        lax.fori_loop(0, 8, _egrp, 0)

    def _out_ref(t):
        u = ubase + t
        s = u // _NBLK
        bhi = u % _NBLK
        return out_hbm.at[s, :, bhi]

    def _start_store(t, b):
        pltpu.make_async_copy(tbuf[b], _out_ref(t), ssem[b]).start()

    def _wait_store(t, b):
        pltpu.make_async_copy(tbuf[b], _out_ref(t), ssem[b]).wait()

    for b in range(_NBUF):
        _start_gather(b, b)

    def _group(g, carry):
        for b in range(_NBUF):
            t = g * _NBUF + b
            _wait_gather(t, b)

            @pl.when(g > 0)
            def _():
                _wait_store(t - _NBUF, b)

            _transpose(b)
            _start_store(t, b)

            @pl.when(g < _NGRP - 1)
            def _():
                _start_gather(t + _NBUF, b)
        return carry

    lax.fori_loop(0, _NGRP, _group, 0)

    for b in range(_NBUF):
        _wait_store((_NGRP - 1) * _NBUF + b, b)


def kernel(vocab_word_idx, vocab_embedding_table):
    idx = (vocab_word_idx.astype(jnp.int32).T
           .reshape(_SEQ, _NBLK, _CHUNK)
           .reshape(_NW, _UPW, _CHUNK))
    out = _sc_gather(idx, vocab_embedding_table)
    return jnp.transpose(out, (2, 4, 0, 1, 3)).reshape(_BATCH, _SEQ, _EMBED)


# speedup vs baseline: 1.8319x; 1.8319x over previous
import functools

import jax
import jax.numpy as jnp
from jax import lax
from jax.experimental import pallas as pl
from jax.experimental.pallas import tpu as pltpu
from jax.experimental.pallas import tpu_sc as plsc

_VOCAB = 1000000
_EMBED = 64
_BATCH = 16384
_SEQ = 50

_NC = 2
_NS = 16
_NW = _NC * _NS
_CHUNK = 128
_NBLK = _BATCH // _CHUNK
_UNITS = _SEQ * _NBLK
_UPW = _UNITS // _NW
_NBUF = 4
_NGRP = _UPW // _NBUF

_mesh = plsc.VectorSubcoreMesh(core_axis_name="c", subcore_axis_name="s")


@functools.partial(
    pl.kernel,
    mesh=_mesh,
    out_type=jax.ShapeDtypeStruct((_SEQ, 8, _NBLK, 8, _CHUNK), jnp.float32),
    scratch_types=(
        [pltpu.VMEM((_UPW, _CHUNK), jnp.int32)]
        + [pltpu.VMEM((_CHUNK, _EMBED), jnp.float32)] * _NBUF
        + [pltpu.VMEM((_EMBED, _CHUNK), jnp.float32)] * _NBUF
        + [pltpu.SemaphoreType.DMA] * _NBUF
        + [pltpu.SemaphoreType.DMA] * _NBUF
    ),
    compiler_params=pltpu.CompilerParams(use_tc_tiling_on_sc=False,
                                         needs_layout_passes=False),
)
def _sc_gather(idx_hbm, table_hbm, out_hbm, idx_v,
               g0, g1, g2, g3, t0, t1, t2, t3,
               gs0, gs1, gs2, gs3, ss0, ss1, ss2, ss3):
    gbuf = (g0, g1, g2, g3)
    tbuf = (t0, t1, t2, t3)
    gsem = (gs0, gs1, gs2, gs3)
    ssem = (ss0, ss1, ss2, ss3)
    wid = lax.axis_index("s") * _NC + lax.axis_index("c")
    ubase = wid * _UPW

    pltpu.sync_copy(idx_hbm.at[wid], idx_v)

    iot = lax.iota(jnp.int32, 16)
    rows16 = [iot + r0 for r0 in range(0, _CHUNK, 16)]

    def _start_gather(t, b):
        pltpu.make_async_copy(table_hbm.at[idx_v.at[t]], gbuf[b],
                              gsem[b]).start()

    def _wait_gather(t, b):
        pltpu.make_async_copy(table_hbm.at[idx_v.at[t]], gbuf[b],
                              gsem[b]).wait()

    def _transpose(b):
        def _j(j, carry):
            diag = (iot + j) & 15
            cols = [diag + e0 for e0 in range(0, _EMBED, 16)]
            for blk in range(32):
                rv = rows16[blk % 8]
                cv = cols[blk // 8]
                v = plsc.load_gather(gbuf[b], [rv, cv])
                plsc.store_scatter(tbuf[b], [cv, rv], v)
            return carry

        lax.fori_loop(0, 16, _j, 0)

    def _store_parts(t, b):
        u = ubase + t
        s = u // _NBLK
        bhi = u % _NBLK
        for ehi in range(8):
            yield (tbuf[b].at[pl.ds(ehi * 8, 8)],
                   out_hbm.at[s, ehi, bhi])

    def _start_store(t, b):
        for src, dst in _store_parts(t, b):
            pltpu.make_async_copy(src, dst, ssem[b]).start()

    def _wait_store(t, b):
        for src, dst in _store_parts(t, b):
            pltpu.make_async_copy(src, dst, ssem[b]).wait()

    for b in range(_NBUF):
        _start_gather(b, b)

    def _group(g, carry):
        for b in range(_NBUF):
            t = g * _NBUF + b
            _wait_gather(t, b)

            @pl.when(g > 0)
            def _():
                _wait_store(t - _NBUF, b)

            _transpose(b)
            _start_store(t, b)

            @pl.when(g < _NGRP - 1)
            def _():
                _start_gather(t + _NBUF, b)
        return carry

    lax.fori_loop(0, _NGRP, _group, 0)

    for b in range(_NBUF):
        _wait_store((_NGRP - 1) * _NBUF + b, b)


def kernel(vocab_word_idx, vocab_embedding_table):
    idx = (vocab_word_idx.astype(jnp.int32).T
           .reshape(_SEQ, _NBLK, _CHUNK)
           .reshape(_NW, _UPW, _CHUNK))
    out = _sc_gather(idx, vocab_embedding_table)
    return jnp.transpose(out, (2, 4, 0, 1, 3)).reshape(_BATCH, _SEQ, _EMBED)


# speedup vs baseline: 1.8350x; 1.0017x over previous
import functools

import jax
import jax.numpy as jnp
from jax import lax
from jax.experimental import pallas as pl
from jax.experimental.pallas import tpu as pltpu
from jax.experimental.pallas import tpu_sc as plsc

_VOCAB = 1000000
_EMBED = 64
_BATCH = 16384
_SEQ = 50

_NC = 2
_NS = 16
_NW = _NC * _NS
_CHUNK = 128
_NBLK = _BATCH // _CHUNK
_UNITS = _SEQ * _NBLK
_UPW = _UNITS // _NW
_NBUF = 4
_NGRP = _UPW // _NBUF

_mesh = plsc.VectorSubcoreMesh(core_axis_name="c", subcore_axis_name="s")


@functools.partial(
    pl.kernel,
    mesh=_mesh,
    out_type=jax.ShapeDtypeStruct((_SEQ, 8, _NBLK, 8, _CHUNK), jnp.float32),
    scratch_types=(
        [pltpu.VMEM((_UPW, _CHUNK), jnp.int32)]
        + [pltpu.VMEM((_CHUNK, _EMBED), jnp.float32)] * _NBUF
        + [pltpu.VMEM((_EMBED, _CHUNK), jnp.float32)] * _NBUF
        + [pltpu.SemaphoreType.DMA] * _NBUF
        + [pltpu.SemaphoreType.DMA] * _NBUF
    ),
    compiler_params=pltpu.CompilerParams(use_tc_tiling_on_sc=False,
                                         needs_layout_passes=False,
                                         disable_bounds_checks=True),
)
def _sc_gather(idx_hbm, table_hbm, out_hbm, idx_v,
               g0, g1, g2, g3, t0, t1, t2, t3,
               gs0, gs1, gs2, gs3, ss0, ss1, ss2, ss3):
    gbuf = (g0, g1, g2, g3)
    tbuf = (t0, t1, t2, t3)
    gsem = (gs0, gs1, gs2, gs3)
    ssem = (ss0, ss1, ss2, ss3)
    wid = lax.axis_index("s") * _NC + lax.axis_index("c")
    ubase = wid * _UPW

    pltpu.sync_copy(idx_hbm.at[wid], idx_v)

    iot = lax.iota(jnp.int32, 16)
    rows16 = [iot + r0 for r0 in range(0, _CHUNK, 16)]

    def _start_gather(t, b):
        pltpu.make_async_copy(table_hbm.at[idx_v.at[t]], gbuf[b],
                              gsem[b]).start()

    def _wait_gather(t, b):
        pltpu.make_async_copy(table_hbm.at[idx_v.at[t]], gbuf[b],
                              gsem[b]).wait()

    def _transpose(b):
        def _j(j, carry):
            diag = (iot + j) & 15
            cols = [diag + e0 for e0 in range(0, _EMBED, 16)]
            for blk in range(32):
                rv = rows16[blk % 8]
                cv = cols[blk // 8]
                v = plsc.load_gather(gbuf[b], [rv, cv])
                plsc.store_scatter(tbuf[b], [cv, rv], v)
            return carry

        lax.fori_loop(0, 16, _j, 0)

    def _store_parts(t, b):
        u = ubase + t
        s = u // _NBLK
        bhi = u % _NBLK
        for ehi in range(8):
            yield (tbuf[b].at[pl.ds(ehi * 8, 8)],
                   out_hbm.at[s, ehi, bhi])

    def _start_store(t, b):
        for src, dst in _store_parts(t, b):
            pltpu.make_async_copy(src, dst, ssem[b]).start()

    def _wait_store(t, b):
        for src, dst in _store_parts(t, b):
            pltpu.make_async_copy(src, dst, ssem[b]).wait()

    for b in range(_NBUF):
        _start_gather(b, b)

    def _group(g, carry):
        for b in range(_NBUF):
            t = g * _NBUF + b
            _wait_gather(t, b)

            @pl.when(g > 0)
            def _():
                _wait_store(t - _NBUF, b)

            _transpose(b)
            _start_store(t, b)

            @pl.when(g < _NGRP - 1)
            def _():
                _start_gather(t + _NBUF, b)
        return carry

    lax.fori_loop(0, _NGRP, _group, 0)

    for b in range(_NBUF):
        _wait_store((_NGRP - 1) * _NBUF + b, b)


def kernel(vocab_word_idx, vocab_embedding_table):
    idx = (vocab_word_idx.astype(jnp.int32).T
           .reshape(_SEQ, _NBLK, _CHUNK)
           .reshape(_NW, _UPW, _CHUNK))
    out = _sc_gather(idx, vocab_embedding_table)
    return jnp.transpose(out, (2, 4, 0, 1, 3)).reshape(_BATCH, _SEQ, _EMBED)


# speedup vs baseline: 1.9545x; 1.0651x over previous
import functools

import jax
import jax.numpy as jnp
from jax import lax
from jax.experimental import pallas as pl
from jax.experimental.pallas import tpu as pltpu
from jax.experimental.pallas import tpu_sc as plsc

_VOCAB = 1000000
_EMBED = 64
_BATCH = 16384
_SEQ = 50

_NC = 2
_NS = 16
_NW = _NC * _NS
_CHUNK = 128
_NBLK = _BATCH // _CHUNK
_UNITS = _SEQ * _NBLK
_UPW = _UNITS // _NW
_NBUF = 4
_NGRP = _UPW // _NBUF

_mesh = plsc.VectorSubcoreMesh(core_axis_name="c", subcore_axis_name="s")


@functools.partial(
    pl.kernel,
    mesh=_mesh,
    out_type=jax.ShapeDtypeStruct((_SEQ, 8, _NBLK, 8, _CHUNK), jnp.float32),
    scratch_types=(
        [pltpu.VMEM((_UPW, _CHUNK), jnp.int32)]
        + [pltpu.VMEM((_CHUNK, 128), jnp.float32)] * _NBUF
        + [pltpu.VMEM((_EMBED, _CHUNK), jnp.float32)] * _NBUF
        + [pltpu.SemaphoreType.DMA] * _NBUF
        + [pltpu.SemaphoreType.DMA] * _NBUF
    ),
    compiler_params=pltpu.CompilerParams(use_tc_tiling_on_sc=False,
                                         needs_layout_passes=False,
                                         disable_bounds_checks=True),
)
def _sc_gather(idx_hbm, table_hbm, out_hbm, idx_v,
               g0, g1, g2, g3, t0, t1, t2, t3,
               gs0, gs1, gs2, gs3, ss0, ss1, ss2, ss3):
    gbuf = (g0, g1, g2, g3)
    tbuf = (t0, t1, t2, t3)
    gsem = (gs0, gs1, gs2, gs3)
    ssem = (ss0, ss1, ss2, ss3)
    wid = lax.axis_index("s") * _NC + lax.axis_index("c")
    ubase = wid * _UPW

    pltpu.sync_copy(idx_hbm.at[wid], idx_v)

    iot = lax.iota(jnp.int32, 16)
    rows16 = [iot + r0 for r0 in range(0, _CHUNK, 16)]

    def _start_gather(t, b):
        pltpu.make_async_copy(table_hbm.at[idx_v.at[t]], gbuf[b],
                              gsem[b]).start()

    def _wait_gather(t, b):
        pltpu.make_async_copy(table_hbm.at[idx_v.at[t]], gbuf[b],
                              gsem[b]).wait()

    def _transpose(b):
        def _j(j, carry):
            diag = (iot + j) & 15
            cols = [diag + e0 for e0 in range(0, _EMBED, 16)]
            for blk in range(32):
                rv = rows16[blk % 8]
                cv = cols[blk // 8]
                v = plsc.load_gather(gbuf[b], [rv, cv])
                plsc.store_scatter(tbuf[b], [cv, rv], v)
            return carry

        lax.fori_loop(0, 16, _j, 0)

    def _store_parts(t, b):
        u = ubase + t
        s = u // _NBLK
        bhi = u % _NBLK
        for ehi in range(8):
            yield (tbuf[b].at[pl.ds(ehi * 8, 8)],
                   out_hbm.at[s, ehi, bhi])

    def _start_store(t, b):
        for src, dst in _store_parts(t, b):
            pltpu.make_async_copy(src, dst, ssem[b]).start()

    def _wait_store(t, b):
        for src, dst in _store_parts(t, b):
            pltpu.make_async_copy(src, dst, ssem[b]).wait()

    for b in range(_NBUF):
        _start_gather(b, b)

    def _group(g, carry):
        for b in range(_NBUF):
            t = g * _NBUF + b
            _wait_gather(t, b)

            @pl.when(g > 0)
            def _():
                _wait_store(t - _NBUF, b)

            _transpose(b)
            _start_store(t, b)

            @pl.when(g < _NGRP - 1)
            def _():
                _start_gather(t + _NBUF, b)
        return carry

    lax.fori_loop(0, _NGRP, _group, 0)

    for b in range(_NBUF):
        _wait_store((_NGRP - 1) * _NBUF + b, b)


def kernel(vocab_word_idx, vocab_embedding_table):
    idx = (vocab_word_idx.astype(jnp.int32).T
           .reshape(_SEQ, _NBLK, _CHUNK)
           .reshape(_NW, _UPW, _CHUNK))
    table_pad = jnp.pad(vocab_embedding_table, ((0, 0), (0, 64)))
    out = _sc_gather(idx, table_pad)
    return jnp.transpose(out, (2, 4, 0, 1, 3)).reshape(_BATCH, _SEQ, _EMBED)
